# baseline (speedup 1.0000x reference)
import functools

import jax
import jax.numpy as jnp
from jax import lax
from jax.experimental import pallas as pl
from jax.experimental.pallas import tpu as pltpu
from jax.experimental.pallas import tpu_sc as plsc

_VOCAB = 1000000
_EMB = 64
_B = 4096
_L = 200
_N = _B * _L

_NW = 32
_PER_W = _N // _NW
_CHUNK = 128
_NCH = _PER_W // _CHUNK
_NBUF = 4


def _make_emb_kernel():
    mesh = plsc.VectorSubcoreMesh(core_axis_name="c", subcore_axis_name="s")

    @functools.partial(
        pl.kernel,
        mesh=mesh,
        out_type=jax.ShapeDtypeStruct((_N, _EMB), jnp.float32),
        compiler_params=pltpu.CompilerParams(use_tc_tiling_on_sc=False),
        scratch_types=[
            pltpu.VMEM((_NCH, _CHUNK), jnp.int32),
            *[pltpu.VMEM((_CHUNK, _EMB), jnp.float32) for _ in range(_NBUF)],
            pltpu.SemaphoreType.DMA,
        ],
    )
    def emb(idx_hbm, table_hbm, out_hbm, idx_v, b0, b1, b2, b3, gsem):
        bufs = [b0, b1, b2, b3]
        wid = lax.axis_index("s") * 2 + lax.axis_index("c")
        base = wid * _PER_W
        pltpu.sync_copy(idx_hbm.at[pl.ds(wid * _NCH, _NCH)], idx_v)

        @pl.loop(0, _NCH, step=_NBUF)
        def group(g):
            handles = []
            for b in range(_NBUF):
                handles.append(
                    pltpu.async_copy(table_hbm.at[idx_v.at[g + b]], bufs[b], gsem)
                )
            for h in handles:
                h.wait()
            for b in range(_NBUF):
                pltpu.sync_copy(
                    bufs[b], out_hbm.at[pl.ds(base + (g + b) * _CHUNK, _CHUNK)]
                )

    return emb


_emb = _make_emb_kernel()


def kernel(x, weight):
    idx = x.reshape(_N // _CHUNK, _CHUNK).astype(jnp.int32)
    out = _emb(idx, weight)
    return out.reshape(_B, _L, _EMB)


# speedup vs baseline: 1.0320x; 1.0320x over previous
import functools

import jax
import jax.numpy as jnp
from jax import lax
from jax.experimental import pallas as pl
from jax.experimental.pallas import tpu as pltpu
from jax.experimental.pallas import tpu_sc as plsc

_VOCAB = 1000000
_EMB = 64
_B = 4096
_L = 200
_N = _B * _L

_NW = 32
_PER_W = _N // _NW
_CHUNK = 128
_NCH = _PER_W // _CHUNK
_NBUF = 8
_LEAD = 4


def _make_emb_kernel():
    mesh = plsc.VectorSubcoreMesh(core_axis_name="c", subcore_axis_name="s")

    @functools.partial(
        pl.kernel,
        mesh=mesh,
        out_type=jax.ShapeDtypeStruct((_N, _EMB), jnp.float32),
        compiler_params=pltpu.CompilerParams(use_tc_tiling_on_sc=False),
        scratch_types=[
            pltpu.VMEM((_NCH, _CHUNK), jnp.int32),
            *[pltpu.VMEM((_CHUNK, _EMB), jnp.float32) for _ in range(_NBUF)],
            *[pltpu.SemaphoreType.DMA for _ in range(2 * _NBUF)],
        ],
    )
    def emb(idx_hbm, table_hbm, out_hbm, idx_v, *rest):
        bufs = rest[:_NBUF]
        gsems = rest[_NBUF:2 * _NBUF]
        wsems = rest[2 * _NBUF:]
        wid = lax.axis_index("s") * 2 + lax.axis_index("c")
        base = wid * _PER_W
        pltpu.sync_copy(idx_hbm.at[pl.ds(wid * _NCH, _NCH)], idx_v)

        for b in range(_LEAD):
            pltpu.async_copy(table_hbm.at[idx_v.at[b]], bufs[b], gsems[b])

        @pl.loop(0, _NCH, step=_NBUF)
        def group(g):
            for b in range(_NBUF):
                j = g + b
                nb = (b + _LEAD) % _NBUF
                pltpu.make_async_copy(
                    table_hbm.at[idx_v.at[0]], bufs[b], gsems[b]
                ).wait()
                pltpu.async_copy(
                    bufs[b], out_hbm.at[pl.ds(base + j * _CHUNK, _CHUNK)],
                    wsems[b],
                )
                @pl.when(jnp.logical_and(j >= _LEAD, j + _LEAD < _NCH))
                def _():
                    pltpu.make_async_copy(
                        bufs[nb], out_hbm.at[pl.ds(0, _CHUNK)], wsems[nb]
                    ).wait()

                @pl.when(j + _LEAD < _NCH)
                def _():
                    pltpu.async_copy(
                        table_hbm.at[idx_v.at[j + _LEAD]], bufs[nb], gsems[nb]
                    )

        for b in range(_NBUF):
            pltpu.make_async_copy(
                bufs[b], out_hbm.at[pl.ds(0, _CHUNK)], wsems[b]
            ).wait()

    return emb


_emb = _make_emb_kernel()


def kernel(x, weight):
    idx = x.reshape(_N // _CHUNK, _CHUNK).astype(jnp.int32)
    out = _emb(idx, weight)
    return out.reshape(_B, _L, _EMB)
